# baseline (speedup 1.0000x reference)
import functools

import jax
import jax.numpy as jnp
from jax import lax
from jax.experimental import pallas as pl
from jax.experimental.pallas import tpu as pltpu
from jax.experimental.pallas import tpu_sc as plsc

_POST = 500
_PRE = 4096
_IOU_T = 0.7
_SCORE_T = 0.1
_BLK = 128
_NBLK = _PRE // _BLK
_OUT_ROWS = 512
_OUT_COLS = 8


_NP = 20480
_JBLK = 128
_CBLK = 128


def _fkey(s):
    k = jax.lax.bitcast_convert_type(s, jnp.int32)
    return jnp.where(k < 0, k ^ jnp.int32(0x7FFFFFFF), k)


def _topk_body(scol_ref, sraw_ref, rank_ref, msr_ref):
    sraw = sraw_ref[0:1, :]
    msr = jnp.where(sraw > _SCORE_T, sraw, -1.0)
    kr = _fkey(msr)
    idxr = jax.lax.broadcasted_iota(jnp.int32, (1, _NP), 1)

    def rank_step(jc, acc):
        sc = scol_ref[pl.ds(jc * _JBLK, _JBLK), 0:1]
        msc = jnp.where(sc > _SCORE_T, sc, -1.0)
        kc = _fkey(msc)
        idxc = jc * _JBLK + jax.lax.broadcasted_iota(
            jnp.int32, (_JBLK, 1), 0)
        prec = (kc > kr) | ((kc == kr) & (idxc < idxr))
        return acc + jnp.sum(jnp.where(prec, 1.0, 0.0), axis=0,
                             keepdims=True)

    rank = jax.lax.fori_loop(0, _NP // _JBLK, rank_step,
                             jnp.zeros((1, _NP), jnp.float32))
    rank_ref[0:1, :] = rank.astype(jnp.int32)
    msr_ref[0:1, :] = msr


def _rank_core(scol, sraw_row):
    return pl.pallas_call(
        _topk_body,
        out_shape=[jax.ShapeDtypeStruct((1, _NP), jnp.int32),
                   jax.ShapeDtypeStruct((1, _NP), jnp.float32)],
    )(scol, sraw_row)


_NW = 32
_RPW = _NP // _NW
_CJUNK = _PRE + _NW


def _sc_scatter_body(rank_hbm, d_hbm, c_hbm, idx_v, d_v, sem):
    wid = lax.axis_index("s") * 2 + lax.axis_index("c")
    nrow = _RPW // 128
    pltpu.sync_copy(rank_hbm.at[wid], idx_v)
    pltpu.sync_copy(d_hbm.at[pl.ds(wid * _RPW, _RPW)], d_v)
    junk = _PRE + wid
    for r in range(nrow):
        for cc in range(8):
            sl = idx_v[r, pl.ds(cc * 16, 16)]
            idx_v[r, pl.ds(cc * 16, 16)] = jnp.where(sl < _PRE, sl, junk)
    for r in range(nrow):
        pltpu.async_copy(d_v.at[pl.ds(r * 128, 128)],
                         c_hbm.at[idx_v.at[r]], sem).wait()


def _sc_scatter(rank2d, d):
    mesh = plsc.VectorSubcoreMesh(core_axis_name="c", subcore_axis_name="s")
    k = functools.partial(
        pl.kernel,
        mesh=mesh,
        out_type=jax.ShapeDtypeStruct((_CJUNK, 128), jnp.float32),
        scratch_types=[
            pltpu.VMEM((_RPW // 128, 128), jnp.int32),
            pltpu.VMEM((_RPW, 128), jnp.float32),
            pltpu.SemaphoreType.DMA,
        ],
    )(_sc_scatter_body)
    return k(rank2d, d)


def _shift_right_lanes(x, d):
    return jnp.concatenate([jnp.zeros((x.shape[0], d), x.dtype), x[:, :-d]], axis=1)


def _nms_body(boxes_col_ref, boxes_row_ref, scores_row_ref, out_ref, sup_ref):
    x1r = boxes_row_ref[0:1, :]
    y1r = boxes_row_ref[1:2, :]
    x2r = boxes_row_ref[2:3, :]
    y2r = boxes_row_ref[3:4, :]
    area_row = jnp.clip(x2r - x1r, 0.0) * jnp.clip(y2r - y1r, 0.0)

    scores_row = scores_row_ref[0:1, :]
    lane = jax.lax.broadcasted_iota(jnp.int32, (1, _PRE), 1)
    keep = scores_row > _SCORE_T

    boxes_col = boxes_col_ref[:, :]

    for kb in range(_NBLK):
        cs = kb * _BLK
        ce = cs + _BLK
        bx1 = boxes_col[cs:ce, 0:1]
        by1 = boxes_col[cs:ce, 1:2]
        bx2 = boxes_col[cs:ce, 2:3]
        by2 = boxes_col[cs:ce, 3:4]
        barea = jnp.clip(bx2 - bx1, 0.0) * jnp.clip(by2 - by1, 0.0)

        xx1 = jnp.maximum(bx1, x1r)
        yy1 = jnp.maximum(by1, y1r)
        xx2 = jnp.minimum(bx2, x2r)
        yy2 = jnp.minimum(by2, y2r)
        inter = jnp.clip(xx2 - xx1, 0.0) * jnp.clip(yy2 - yy1, 0.0)
        union = barea + area_row - inter
        iou = inter / jnp.maximum(union, 1e-8)
        supf = jnp.where(iou > _IOU_T, 1.0, 0.0)

        sup_ref[:, :] = supf[:, cs:ce]
        blane = jax.lax.broadcasted_iota(jnp.int32, (1, _BLK), 1)
        keep_b0 = jnp.where(keep[:, cs:ce], 1.0, 0.0)

        def intra(j, kb_):
            row = sup_ref[pl.ds(j, 1), :]
            kj = jnp.sum(jnp.where(blane == j, kb_, 0.0), axis=1,
                         keepdims=True)
            sup = (row > 0.5) & (blane > j) & (kj > 0.5)
            return jnp.where(sup, 0.0, kb_)

        kbf = jax.lax.fori_loop(0, _BLK, intra, keep_b0)
        keep_b = kbf > 0.5

        cnt = jax.lax.dot_general(
            kbf, supf, (((1,), (0,)), ((), ())),
            preferred_element_type=jnp.float32)
        crossed = (cnt > 0.5) & (lane >= ce)
        pieces = []
        if cs > 0:
            pieces.append(keep[:, :cs])
        pieces.append(keep_b)
        if ce < _PRE:
            pieces.append(keep[:, ce:])
        keep = jnp.concatenate(pieces, axis=1) if len(pieces) > 1 else pieces[0]
        keep = keep & jnp.logical_not(crossed)

    kept_scores = jnp.where(keep, scores_row, -1.0)

    keepf = jnp.where(keep, 1.0, 0.0)
    csum = keepf
    d = 1
    while d < _PRE:
        csum = csum + _shift_right_lanes(csum, d)
        d *= 2
    total = csum[:, _PRE - 1:_PRE]
    lanef = lane.astype(jnp.float32)
    pos = jnp.where(keep, csum - 1.0, total + lanef - csum)

    rows = jax.lax.broadcasted_iota(jnp.int32, (_OUT_ROWS, 1), 0)
    onehot = jnp.where(rows == pos.astype(jnp.int32), 1.0, 0.0)

    out_boxes = jax.lax.dot_general(
        onehot, boxes_col, (((1,), (0,)), ((), ())),
        preferred_element_type=jnp.float32)
    out_scores = jnp.sum(onehot * kept_scores, axis=1, keepdims=True)
    pad = jnp.zeros((_OUT_ROWS, _OUT_COLS - 5), jnp.float32)
    out_ref[:, :] = jnp.concatenate([out_boxes, out_scores, pad], axis=1)


def _nms_core(boxes_col, boxes_row, scores_row):
    return pl.pallas_call(
        _nms_body,
        out_shape=jax.ShapeDtypeStruct((_OUT_ROWS, _OUT_COLS), jnp.float32),
        scratch_shapes=[pltpu.VMEM((_BLK, _BLK), jnp.float32)],
    )(boxes_col, boxes_row, scores_row)


def kernel(batch_box_preds, batch_cls_preds):
    n = batch_box_preds.shape[2]
    boxes = batch_box_preds[0].T
    scores = jnp.max(batch_cls_preds[0], axis=0)
    bc = jnp.pad(boxes, ((0, _NP - n), (0, 0)))
    sc = jnp.pad(scores, (0, _NP - n), constant_values=-2.0)
    rank, msr = _rank_core(sc[:, None], sc[None, :])
    d = jnp.concatenate(
        [bc, msr.reshape(_NP, 1), jnp.zeros((_NP, 123), jnp.float32)],
        axis=1)
    c = _sc_scatter(rank.reshape(_NW, _RPW // 128, 128), d)[:_PRE]
    top_boxes = c[:, :4]
    out = _nms_core(top_boxes, top_boxes.T, c[:, 4][None, :])
    return out[:_POST, :6]


# speedup vs baseline: 2.3057x; 2.3057x over previous
import functools

import jax
import jax.numpy as jnp
from jax import lax
from jax.experimental import pallas as pl
from jax.experimental.pallas import tpu as pltpu
from jax.experimental.pallas import tpu_sc as plsc

_POST = 500
_PRE = 4096
_IOU_T = 0.7
_SCORE_T = 0.1
_BLK = 128
_NBLK = _PRE // _BLK
_OUT_ROWS = 512
_OUT_COLS = 8
_NP = 20480
_NW = 32
_CJUNK = _PRE + _NW
_KLO = -1073741826
_KHI = 1065353216


def _fkey(s):
    k = jax.lax.bitcast_convert_type(s, jnp.int32)
    return jnp.where(k < 0, k ^ jnp.int32(0x7FFFFFFF), k)


def _shift_right_lanes(x, d):
    return jnp.concatenate([jnp.zeros((x.shape[0], d), x.dtype), x[:, :-d]],
                           axis=1)


def _cumsum_lanes(x):
    n = x.shape[1]
    d = 1
    while d < n:
        x = x + _shift_right_lanes(x, d)
        d *= 2
    return x


def _select_body(scol_ref, sraw_ref, pos_ref, msr_ref):
    sraw = sraw_ref[0:1, :]
    msr = jnp.where(sraw > _SCORE_T, sraw, -1.0)
    kr = _fkey(msr)

    scol = scol_ref[:, 0:1]
    mscol = jnp.where(scol > _SCORE_T, scol, -1.0)
    kcol = _fkey(mscol)

    ones_row = jnp.ones((1, _NP), jnp.float32)
    m = jax.lax.broadcasted_iota(jnp.int32, (1, 128), 1)
    lo = jnp.full((), _KLO, jnp.int32)
    hi = jnp.full((), _KHI, jnp.int32)
    for _ in range(5):
        step = jnp.maximum(jnp.int32(1), (hi - lo + 127) // 128)
        t = jnp.minimum(lo + (m + 1) * step, hi)
        cmpf = jnp.where(kcol > t, 1.0, 0.0)
        counts = jax.lax.dot_general(
            ones_row, cmpf, (((1,), (0,)), ((), ())),
            preferred_element_type=jnp.float32)
        selm = counts >= float(_PRE)
        lo = jnp.max(jnp.where(selm, t, lo))
        hi = jnp.min(jnp.where(selm, hi, t))
    vstar = hi

    gt = kr > vstar
    ngt = jnp.sum(jnp.where(gt, 1.0, 0.0))
    tief = jnp.where(kr == vstar, 1.0, 0.0)
    texcl = _cumsum_lanes(tief) - tief
    sel = gt | ((kr == vstar) & (ngt + texcl < float(_PRE)))
    self_ = jnp.where(sel, 1.0, 0.0)
    sexcl = _cumsum_lanes(self_) - self_
    pos = jnp.where(sel, sexcl, float(_PRE))
    pos_ref[0:1, :] = pos.astype(jnp.int32)
    msr_ref[0:1, :] = msr


def _select_core(scol, sraw_row):
    return pl.pallas_call(
        _select_body,
        out_shape=[jax.ShapeDtypeStruct((1, _NP), jnp.int32),
                   jax.ShapeDtypeStruct((1, _NP), jnp.float32)],
    )(scol, sraw_row)


def _rank2_body(c_ref, msrow_ref, idxrow_ref, rank_ref):
    krow = _fkey(msrow_ref[0:1, :])
    idxrow = idxrow_ref[0:1, :]

    def step(jc, acc):
        sc = c_ref[pl.ds(jc * _BLK, _BLK), 4:5]
        kc = _fkey(sc)
        idxc = c_ref[pl.ds(jc * _BLK, _BLK), 5:6]
        prec = (kc > krow) | ((kc == krow) & (idxc < idxrow))
        return acc + jnp.sum(jnp.where(prec, 1.0, 0.0), axis=0,
                             keepdims=True)

    rank = jax.lax.fori_loop(0, _NBLK, step,
                             jnp.zeros((1, _PRE), jnp.float32))
    rank_ref[0:1, :] = rank.astype(jnp.int32)


def _rank2_core(c, msrow, idxrow):
    return pl.pallas_call(
        _rank2_body,
        out_shape=jax.ShapeDtypeStruct((1, _PRE), jnp.int32),
    )(c, msrow, idxrow)


def _make_sc_scatter(nrows):
    rpw = nrows // _NW
    nrow = rpw // 128

    def body(rank_hbm, d_hbm, c_hbm, idx_v, d_v, sem):
        wid = lax.axis_index("s") * 2 + lax.axis_index("c")
        pltpu.sync_copy(rank_hbm.at[wid], idx_v)
        pltpu.sync_copy(d_hbm.at[pl.ds(wid * rpw, rpw)], d_v)
        junk = _PRE + wid
        for r in range(nrow):
            for cc in range(8):
                sl = idx_v[r, pl.ds(cc * 16, 16)]
                idx_v[r, pl.ds(cc * 16, 16)] = jnp.where(sl < _PRE, sl, junk)
        for r in range(nrow):
            pltpu.async_copy(d_v.at[pl.ds(r * 128, 128)],
                             c_hbm.at[idx_v.at[r]], sem).wait()

    mesh = plsc.VectorSubcoreMesh(core_axis_name="c", subcore_axis_name="s")
    k = functools.partial(
        pl.kernel,
        mesh=mesh,
        out_type=jax.ShapeDtypeStruct((_CJUNK, 128), jnp.float32),
        scratch_types=[
            pltpu.VMEM((nrow, 128), jnp.int32),
            pltpu.VMEM((rpw, 128), jnp.float32),
            pltpu.SemaphoreType.DMA,
        ],
    )(body)

    def run(rank_row, d):
        return k(rank_row.reshape(_NW, nrow, 128), d)[:_PRE]

    return run


_sc_compact = _make_sc_scatter(_NP)
_sc_sort = _make_sc_scatter(_PRE)


def _nms_body(boxes_col_ref, boxes_row_ref, scores_row_ref, out_ref, sup_ref):
    x1r = boxes_row_ref[0:1, :]
    y1r = boxes_row_ref[1:2, :]
    x2r = boxes_row_ref[2:3, :]
    y2r = boxes_row_ref[3:4, :]
    area_row = jnp.clip(x2r - x1r, 0.0) * jnp.clip(y2r - y1r, 0.0)

    scores_row = scores_row_ref[0:1, :]
    lane = jax.lax.broadcasted_iota(jnp.int32, (1, _PRE), 1)
    keep = scores_row > _SCORE_T

    boxes_col = boxes_col_ref[:, :]

    for kb in range(_NBLK):
        cs = kb * _BLK
        ce = cs + _BLK
        bx1 = boxes_col[cs:ce, 0:1]
        by1 = boxes_col[cs:ce, 1:2]
        bx2 = boxes_col[cs:ce, 2:3]
        by2 = boxes_col[cs:ce, 3:4]
        barea = jnp.clip(bx2 - bx1, 0.0) * jnp.clip(by2 - by1, 0.0)

        xx1 = jnp.maximum(bx1, x1r)
        yy1 = jnp.maximum(by1, y1r)
        xx2 = jnp.minimum(bx2, x2r)
        yy2 = jnp.minimum(by2, y2r)
        inter = jnp.clip(xx2 - xx1, 0.0) * jnp.clip(yy2 - yy1, 0.0)
        union = barea + area_row - inter
        iou = inter / jnp.maximum(union, 1e-8)
        supf = jnp.where(iou > _IOU_T, 1.0, 0.0)

        sup_ref[:, :] = supf[:, cs:ce]
        blane = jax.lax.broadcasted_iota(jnp.int32, (1, _BLK), 1)
        keep_b0 = jnp.where(keep[:, cs:ce], 1.0, 0.0)

        def intra(j, kb_):
            row = sup_ref[pl.ds(j, 1), :]
            kj = jnp.sum(jnp.where(blane == j, kb_, 0.0), axis=1,
                         keepdims=True)
            sup = (row > 0.5) & (blane > j) & (kj > 0.5)
            return jnp.where(sup, 0.0, kb_)

        kbf = jax.lax.fori_loop(0, _BLK, intra, keep_b0)
        keep_b = kbf > 0.5

        cnt = jax.lax.dot_general(
            kbf, supf, (((1,), (0,)), ((), ())),
            preferred_element_type=jnp.float32)
        crossed = (cnt > 0.5) & (lane >= ce)
        pieces = []
        if cs > 0:
            pieces.append(keep[:, :cs])
        pieces.append(keep_b)
        if ce < _PRE:
            pieces.append(keep[:, ce:])
        keep = jnp.concatenate(pieces, axis=1) if len(pieces) > 1 else pieces[0]
        keep = keep & jnp.logical_not(crossed)

    kept_scores = jnp.where(keep, scores_row, -1.0)

    keepf = jnp.where(keep, 1.0, 0.0)
    csum = _cumsum_lanes(keepf)
    total = csum[:, _PRE - 1:_PRE]
    lanef = lane.astype(jnp.float32)
    pos = jnp.where(keep, csum - 1.0, total + lanef - csum)

    rows = jax.lax.broadcasted_iota(jnp.int32, (_OUT_ROWS, 1), 0)
    onehot = jnp.where(rows == pos.astype(jnp.int32), 1.0, 0.0)

    out_boxes = jax.lax.dot_general(
        onehot, boxes_col, (((1,), (0,)), ((), ())),
        preferred_element_type=jnp.float32)
    out_scores = jnp.sum(onehot * kept_scores, axis=1, keepdims=True)
    pad = jnp.zeros((_OUT_ROWS, _OUT_COLS - 5), jnp.float32)
    out_ref[:, :] = jnp.concatenate([out_boxes, out_scores, pad], axis=1)


def _nms_core(boxes_col, boxes_row, scores_row):
    return pl.pallas_call(
        _nms_body,
        out_shape=jax.ShapeDtypeStruct((_OUT_ROWS, _OUT_COLS), jnp.float32),
        scratch_shapes=[pltpu.VMEM((_BLK, _BLK), jnp.float32)],
    )(boxes_col, boxes_row, scores_row)


def kernel(batch_box_preds, batch_cls_preds):
    n = batch_box_preds.shape[2]
    boxes = batch_box_preds[0].T
    scores = jnp.max(batch_cls_preds[0], axis=0)
    bc = jnp.pad(boxes, ((0, _NP - n), (0, 0)))
    sc = jnp.pad(scores, (0, _NP - n), constant_values=-2.0)
    pos, msr = _select_core(sc[:, None], sc[None, :])
    idxf = jnp.arange(_NP, dtype=jnp.float32)[:, None]
    d = jnp.concatenate(
        [bc, msr.reshape(_NP, 1), idxf, jnp.zeros((_NP, 122), jnp.float32)],
        axis=1)
    ci = _sc_compact(pos, d)
    rank2 = _rank2_core(ci, ci[:, 4].reshape(1, _PRE),
                        ci[:, 5].reshape(1, _PRE))
    c = _sc_sort(rank2, ci)
    top_boxes = c[:, :4]
    out = _nms_core(top_boxes, top_boxes.T, c[:, 4][None, :])
    return out[:_POST, :6]


# speedup vs baseline: 4.5650x; 1.9799x over previous
import functools

import jax
import jax.numpy as jnp
from jax import lax
from jax.experimental import pallas as pl
from jax.experimental.pallas import tpu as pltpu
from jax.experimental.pallas import tpu_sc as plsc

_POST = 500
_PRE = 4096
_IOU_T = 0.7
_SCORE_T = 0.1
_BLK = 128
_NBLK = _PRE // _BLK
_OUT_ROWS = 512
_OUT_COLS = 8
_NP = 20480
_NW = 32
_CJUNK = _PRE + _NW
_KLO = -1073741826
_KHI = 1065353216


def _fkey(s):
    k = jax.lax.bitcast_convert_type(s, jnp.int32)
    return jnp.where(k < 0, k ^ jnp.int32(0x7FFFFFFF), k)


def _shift_right_lanes(x, d):
    return jnp.concatenate([jnp.zeros((x.shape[0], d), x.dtype), x[:, :-d]],
                           axis=1)


def _cumsum_lanes(x):
    n = x.shape[1]
    d = 1
    while d < n:
        x = x + _shift_right_lanes(x, d)
        d *= 2
    return x


def _select_body(scol_ref, sraw_ref, pos_ref, msr_ref):
    sraw = sraw_ref[0:1, :]
    msr = jnp.where(sraw > _SCORE_T, sraw, -1.0)
    kr = _fkey(msr)

    cchunk = 2048
    ones_row = jnp.ones((1, cchunk), jnp.float32)
    m = jax.lax.broadcasted_iota(jnp.int32, (1, 128), 1)
    lo = jnp.full((), _KLO, jnp.int32)
    hi = jnp.full((), _KHI, jnp.int32)
    for _ in range(5):
        step = jnp.maximum(jnp.int32(1), (hi - lo + 127) // 128)
        t = jnp.minimum(lo + (m + 1) * step, hi)

        def cstep(jc, acc):
            scol = scol_ref[pl.ds(jc * cchunk, cchunk), 0:1]
            mscol = jnp.where(scol > _SCORE_T, scol, -1.0)
            kcol = _fkey(mscol)
            cmpf = jnp.where(kcol > t, 1.0, 0.0)
            return acc + jax.lax.dot_general(
                ones_row, cmpf, (((1,), (0,)), ((), ())),
                preferred_element_type=jnp.float32)

        counts = jax.lax.fori_loop(0, _NP // cchunk, cstep,
                                   jnp.zeros((1, 128), jnp.float32))
        selm = counts >= float(_PRE)
        lo = jnp.max(jnp.where(selm, t, lo))
        hi = jnp.min(jnp.where(selm, hi, t))
    vstar = hi

    gt = kr > vstar
    ngt = jnp.sum(jnp.where(gt, 1.0, 0.0))
    tief = jnp.where(kr == vstar, 1.0, 0.0)
    texcl = _cumsum_lanes(tief) - tief
    sel = gt | ((kr == vstar) & (ngt + texcl < float(_PRE)))
    self_ = jnp.where(sel, 1.0, 0.0)
    sexcl = _cumsum_lanes(self_) - self_
    pos = jnp.where(sel, sexcl, float(_PRE))
    pos_ref[0:1, :] = pos.astype(jnp.int32)
    msr_ref[0:1, :] = msr


def _select_core(scol, sraw_row):
    return pl.pallas_call(
        _select_body,
        out_shape=[jax.ShapeDtypeStruct((1, _NP), jnp.int32),
                   jax.ShapeDtypeStruct((1, _NP), jnp.float32)],
    )(scol, sraw_row)


def _rank2_body(c_ref, msrow_ref, idxrow_ref, rank_ref):
    krow = _fkey(msrow_ref[0:1, :])
    idxrow = idxrow_ref[0:1, :]

    def step(jc, acc):
        sc = c_ref[pl.ds(jc * _BLK, _BLK), 4:5]
        kc = _fkey(sc)
        idxc = c_ref[pl.ds(jc * _BLK, _BLK), 5:6]
        prec = (kc > krow) | ((kc == krow) & (idxc < idxrow))
        return acc + jnp.sum(jnp.where(prec, 1.0, 0.0), axis=0,
                             keepdims=True)

    rank = jax.lax.fori_loop(0, _NBLK, step,
                             jnp.zeros((1, _PRE), jnp.float32))
    rank_ref[0:1, :] = rank.astype(jnp.int32)


def _rank2_core(c, msrow, idxrow):
    return pl.pallas_call(
        _rank2_body,
        out_shape=jax.ShapeDtypeStruct((1, _PRE), jnp.int32),
    )(c, msrow, idxrow)


@functools.lru_cache(maxsize=None)
def _make_sc_scatter(nrows):
    rpw = nrows // _NW
    nrow = rpw // 128

    def body(rank_hbm, d_hbm, c_hbm, idx_v, d_v, sem):
        wid = lax.axis_index("s") * 2 + lax.axis_index("c")
        pltpu.sync_copy(rank_hbm.at[wid], idx_v)
        pltpu.sync_copy(d_hbm.at[pl.ds(wid * rpw, rpw)], d_v)
        junk = _PRE + wid
        for r in range(nrow):
            for cc in range(8):
                sl = idx_v[r, pl.ds(cc * 16, 16)]
                idx_v[r, pl.ds(cc * 16, 16)] = jnp.where(sl < _PRE, sl, junk)
        for r in range(nrow):
            pltpu.async_copy(d_v.at[pl.ds(r * 128, 128)],
                             c_hbm.at[idx_v.at[r]], sem).wait()

    mesh = plsc.VectorSubcoreMesh(core_axis_name="c", subcore_axis_name="s")
    k = functools.partial(
        pl.kernel,
        mesh=mesh,
        out_type=jax.ShapeDtypeStruct((_CJUNK, 128), jnp.float32),
        scratch_types=[
            pltpu.VMEM((nrow, 128), jnp.int32),
            pltpu.VMEM((rpw, 128), jnp.float32),
            pltpu.SemaphoreType.DMA,
        ],
    )(body)

    def run(rank_row, d):
        return k(rank_row.reshape(_NW, nrow, 128), d)[:_PRE]

    return run


def _sc_compact(rank_row, d):
    return _make_sc_scatter(_NP)(rank_row, d)


def _sc_sort(rank_row, d):
    return _make_sc_scatter(_PRE)(rank_row, d)


def _nms_build_body(boxes_col_ref, boxes_row_ref, sup_ref):
    x1r = boxes_row_ref[0:1, :]
    y1r = boxes_row_ref[1:2, :]
    x2r = boxes_row_ref[2:3, :]
    y2r = boxes_row_ref[3:4, :]
    area_row = jnp.clip(x2r - x1r, 0.0) * jnp.clip(y2r - y1r, 0.0)
    lane = jax.lax.broadcasted_iota(jnp.int32, (1, _PRE), 1)
    bb = 64

    def build(kb, _):
        cs = kb * bb
        bx1 = boxes_col_ref[pl.ds(cs, bb), 0:1]
        by1 = boxes_col_ref[pl.ds(cs, bb), 1:2]
        bx2 = boxes_col_ref[pl.ds(cs, bb), 2:3]
        by2 = boxes_col_ref[pl.ds(cs, bb), 3:4]
        barea = jnp.clip(bx2 - bx1, 0.0) * jnp.clip(by2 - by1, 0.0)

        xx1 = jnp.maximum(bx1, x1r)
        yy1 = jnp.maximum(by1, y1r)
        xx2 = jnp.minimum(bx2, x2r)
        yy2 = jnp.minimum(by2, y2r)
        inter = jnp.clip(xx2 - xx1, 0.0) * jnp.clip(yy2 - yy1, 0.0)
        union = barea + area_row - inter
        iou = inter / jnp.maximum(union, 1e-8)
        gj = cs + jax.lax.broadcasted_iota(jnp.int32, (bb, 1), 0)
        sup = (iou > _IOU_T) & (gj < lane)
        sup_ref[pl.ds(cs, bb), :] = jnp.where(sup, 1.0, 0.0).astype(
            jnp.bfloat16)
        return 0

    jax.lax.fori_loop(0, _PRE // bb, build, 0)


def _nms_iter_body(sup_ref, boxes_col_ref, scores_row_ref, out_ref, kvec_ref):
    scores_row = scores_row_ref[0:1, :]
    lane = jax.lax.broadcasted_iota(jnp.int32, (1, _PRE), 1)
    init = scores_row > _SCORE_T
    initf = jnp.where(init, 1.0, 0.0)
    boxes_col = boxes_col_ref[:, :]

    def F(kf):
        kvec_ref[0:1, :] = kf.astype(jnp.bfloat16)

        def mstep(mc, cnt):
            cs = pl.multiple_of(mc * 512, 512)
            lhs = kvec_ref[0:1, pl.ds(cs, 512)]
            rhs = sup_ref[pl.ds(cs, 512), :]
            return cnt + jax.lax.dot_general(
                lhs, rhs, (((1,), (0,)), ((), ())),
                preferred_element_type=jnp.float32)

        cnt = jax.lax.fori_loop(0, 8, mstep,
                                jnp.zeros((1, _PRE), jnp.float32))
        return jnp.where(cnt < 0.5, initf, 0.0)

    def cond(carry):
        lf, uf = carry
        return jnp.any(lf != uf)

    def step(carry):
        lf, uf = carry
        uf2 = F(lf)
        return F(uf2), uf2

    u0 = initf
    l0 = F(u0)
    lf, uf = jax.lax.while_loop(cond, step, (l0, u0))
    keep = uf > 0.5
    kept_scores = jnp.where(keep, scores_row, -1.0)

    keepf = jnp.where(keep, 1.0, 0.0)
    csum = _cumsum_lanes(keepf)
    total = csum[:, _PRE - 1:_PRE]
    lanef = lane.astype(jnp.float32)
    pos = jnp.where(keep, csum - 1.0, total + lanef - csum)

    posi = pos.astype(jnp.int32)
    pad = jnp.zeros((128, _OUT_COLS - 5), jnp.float32)
    for oc in range(_OUT_ROWS // 128):
        rows = oc * 128 + jax.lax.broadcasted_iota(jnp.int32, (128, 1), 0)
        onehot = jnp.where(rows == posi, 1.0, 0.0)
        ob = jax.lax.dot_general(
            onehot, boxes_col, (((1,), (0,)), ((), ())),
            preferred_element_type=jnp.float32)
        os_ = jnp.sum(onehot * kept_scores, axis=1, keepdims=True)
        out_ref[oc * 128:(oc + 1) * 128, :] = jnp.concatenate(
            [ob, os_, pad], axis=1)


def _nms_core(boxes_col, boxes_row, scores_row):
    sup = pl.pallas_call(
        _nms_build_body,
        out_shape=jax.ShapeDtypeStruct((_PRE, _PRE), jnp.bfloat16),
    )(boxes_col, boxes_row)
    return pl.pallas_call(
        _nms_iter_body,
        out_shape=jax.ShapeDtypeStruct((_OUT_ROWS, _OUT_COLS), jnp.float32),
        scratch_shapes=[pltpu.VMEM((1, _PRE), jnp.bfloat16)],
    )(sup, boxes_col, scores_row)


def kernel(batch_box_preds, batch_cls_preds):
    n = batch_box_preds.shape[2]
    boxes = batch_box_preds[0].T
    scores = jnp.max(batch_cls_preds[0], axis=0)
    bc = jnp.pad(boxes, ((0, _NP - n), (0, 0)))
    sc = jnp.pad(scores, (0, _NP - n), constant_values=-2.0)
    pos, msr = _select_core(sc[:, None], sc[None, :])
    idxf = jnp.arange(_NP, dtype=jnp.float32)[:, None]
    d = jnp.concatenate(
        [bc, msr.reshape(_NP, 1), idxf, jnp.zeros((_NP, 122), jnp.float32)],
        axis=1)
    ci = _sc_compact(pos, d)
    rank2 = _rank2_core(ci, ci[:, 4].reshape(1, _PRE),
                        ci[:, 5].reshape(1, _PRE))
    c = _sc_sort(rank2, ci)
    top_boxes = c[:, :4]
    out = _nms_core(top_boxes, top_boxes.T, c[:, 4][None, :])
    return out[:_POST, :6]


# speedup vs baseline: 4.6891x; 1.0272x over previous
import functools

import jax
import jax.numpy as jnp
from jax import lax
from jax.experimental import pallas as pl
from jax.experimental.pallas import tpu as pltpu
from jax.experimental.pallas import tpu_sc as plsc

_POST = 500
_PRE = 4096
_IOU_T = 0.7
_SCORE_T = 0.1
_BLK = 128
_NBLK = _PRE // _BLK
_OUT_ROWS = 512
_OUT_COLS = 8
_NP = 20480
_NW = 32
_CJUNK = _PRE + _NW
_KLO = -1073741826
_KHI = 1065353216


def _fkey(s):
    k = jax.lax.bitcast_convert_type(s, jnp.int32)
    return jnp.where(k < 0, k ^ jnp.int32(0x7FFFFFFF), k)


def _shift_right_lanes(x, d):
    return jnp.concatenate([jnp.zeros((x.shape[0], d), x.dtype), x[:, :-d]],
                           axis=1)


def _cumsum_lanes(x):
    n = x.shape[1]
    d = 1
    while d < n:
        x = x + _shift_right_lanes(x, d)
        d *= 2
    return x


def _select_body(scol_ref, sraw_ref, pos_ref, msr_ref):
    sraw = sraw_ref[0:1, :]
    msr = jnp.where(sraw > _SCORE_T, sraw, -1.0)
    kr = _fkey(msr)

    cchunk = 2048
    ones_row = jnp.ones((1, cchunk), jnp.float32)
    m = jax.lax.broadcasted_iota(jnp.int32, (1, 128), 1)
    lo = jnp.full((), _KLO, jnp.int32)
    hi = jnp.full((), _KHI, jnp.int32)
    for _ in range(5):
        step = jnp.maximum(jnp.int32(1), (hi - lo + 127) // 128)
        t = jnp.minimum(lo + (m + 1) * step, hi)

        def cstep(jc, acc):
            scol = scol_ref[pl.ds(jc * cchunk, cchunk), 0:1]
            mscol = jnp.where(scol > _SCORE_T, scol, -1.0)
            kcol = _fkey(mscol)
            cmpf = jnp.where(kcol > t, 1.0, 0.0)
            return acc + jax.lax.dot_general(
                ones_row, cmpf, (((1,), (0,)), ((), ())),
                preferred_element_type=jnp.float32)

        counts = jax.lax.fori_loop(0, _NP // cchunk, cstep,
                                   jnp.zeros((1, 128), jnp.float32))
        selm = counts >= float(_PRE)
        lo = jnp.max(jnp.where(selm, t, lo))
        hi = jnp.min(jnp.where(selm, hi, t))
    vstar = hi

    gt = kr > vstar
    ngt = jnp.sum(jnp.where(gt, 1.0, 0.0))
    tief = jnp.where(kr == vstar, 1.0, 0.0)
    texcl = _cumsum_lanes(tief) - tief
    sel = gt | ((kr == vstar) & (ngt + texcl < float(_PRE)))
    self_ = jnp.where(sel, 1.0, 0.0)
    sexcl = _cumsum_lanes(self_) - self_
    pos = jnp.where(sel, sexcl, float(_PRE))
    pos_ref[0:1, :] = pos.astype(jnp.int32)
    msr_ref[0:1, :] = msr


def _select_core(scol, sraw_row):
    return pl.pallas_call(
        _select_body,
        out_shape=[jax.ShapeDtypeStruct((1, _NP), jnp.int32),
                   jax.ShapeDtypeStruct((1, _NP), jnp.float32)],
    )(scol, sraw_row)


def _rank2_body(c_ref, msrow_ref, idxrow_ref, rank_ref):
    krow = _fkey(msrow_ref[0:1, :])
    idxrow = idxrow_ref[0:1, :]

    def step(jc, acc):
        sc = c_ref[pl.ds(jc * _BLK, _BLK), 4:5]
        kc = _fkey(sc)
        idxc = c_ref[pl.ds(jc * _BLK, _BLK), 5:6]
        prec = (kc > krow) | ((kc == krow) & (idxc < idxrow))
        return acc + jnp.sum(jnp.where(prec, 1.0, 0.0), axis=0,
                             keepdims=True)

    rank = jax.lax.fori_loop(0, _NBLK, step,
                             jnp.zeros((1, _PRE), jnp.float32))
    rank_ref[0:1, :] = rank.astype(jnp.int32)


def _rank2_core(c, msrow, idxrow):
    return pl.pallas_call(
        _rank2_body,
        out_shape=jax.ShapeDtypeStruct((1, _PRE), jnp.int32),
    )(c, msrow, idxrow)


@functools.lru_cache(maxsize=None)
def _make_sc_scatter(nrows):
    rpw = nrows // _NW
    nrow = rpw // 128

    def body(rank_hbm, d_hbm, c_hbm, idx_v, d_v, sem):
        wid = lax.axis_index("s") * 2 + lax.axis_index("c")
        pltpu.sync_copy(rank_hbm.at[wid], idx_v)
        pltpu.sync_copy(d_hbm.at[pl.ds(wid * rpw, rpw)], d_v)
        junk = _PRE + wid
        for r in range(nrow):
            for cc in range(8):
                sl = idx_v[r, pl.ds(cc * 16, 16)]
                idx_v[r, pl.ds(cc * 16, 16)] = jnp.where(sl < _PRE, sl, junk)
        for r in range(nrow):
            pltpu.async_copy(d_v.at[pl.ds(r * 128, 128)],
                             c_hbm.at[idx_v.at[r]], sem).wait()

    mesh = plsc.VectorSubcoreMesh(core_axis_name="c", subcore_axis_name="s")
    k = functools.partial(
        pl.kernel,
        mesh=mesh,
        out_type=jax.ShapeDtypeStruct((_CJUNK, 128), jnp.float32),
        scratch_types=[
            pltpu.VMEM((nrow, 128), jnp.int32),
            pltpu.VMEM((rpw, 128), jnp.float32),
            pltpu.SemaphoreType.DMA,
        ],
    )(body)

    def run(rank_row, d):
        return k(rank_row.reshape(_NW, nrow, 128), d)[:_PRE]

    return run


def _sc_compact(rank_row, d):
    return _make_sc_scatter(_NP)(rank_row, d)


def _sc_sort(rank_row, d):
    return _make_sc_scatter(_PRE)(rank_row, d)


def _nms_build_body(boxes_col_ref, boxes_row_ref, sup_ref):
    x1r = boxes_row_ref[0:1, :]
    y1r = boxes_row_ref[1:2, :]
    x2r = boxes_row_ref[2:3, :]
    y2r = boxes_row_ref[3:4, :]
    area_row = jnp.clip(x2r - x1r, 0.0) * jnp.clip(y2r - y1r, 0.0)
    lane = jax.lax.broadcasted_iota(jnp.int32, (1, _PRE), 1)
    bb = 128

    def build(kb, _):
        cs = kb * bb
        bx1 = boxes_col_ref[pl.ds(cs, bb), 0:1]
        by1 = boxes_col_ref[pl.ds(cs, bb), 1:2]
        bx2 = boxes_col_ref[pl.ds(cs, bb), 2:3]
        by2 = boxes_col_ref[pl.ds(cs, bb), 3:4]
        barea = jnp.clip(bx2 - bx1, 0.0) * jnp.clip(by2 - by1, 0.0)

        xx1 = jnp.maximum(bx1, x1r)
        yy1 = jnp.maximum(by1, y1r)
        xx2 = jnp.minimum(bx2, x2r)
        yy2 = jnp.minimum(by2, y2r)
        inter = jnp.clip(xx2 - xx1, 0.0) * jnp.clip(yy2 - yy1, 0.0)
        union = barea + area_row - inter
        iou = inter / jnp.maximum(union, 1e-8)
        gj = cs + jax.lax.broadcasted_iota(jnp.int32, (bb, 1), 0)
        sup = (iou > _IOU_T) & (gj < lane)
        sup_ref[pl.ds(cs, bb), :] = jnp.where(sup, 1.0, 0.0).astype(
            jnp.bfloat16)
        return 0

    jax.lax.fori_loop(0, _PRE // bb, build, 0)


def _nms_iter_body(sup_ref, boxes_col_ref, scores_row_ref, out_ref, kvec_ref):
    scores_row = scores_row_ref[0:1, :]
    lane = jax.lax.broadcasted_iota(jnp.int32, (1, _PRE), 1)
    init = scores_row > _SCORE_T
    initf = jnp.where(init, 1.0, 0.0)
    boxes_col = boxes_col_ref[:, :]

    def F(kf):
        kvec_ref[0:1, :] = kf.astype(jnp.bfloat16)

        def mstep(mc, cnt):
            cs = pl.multiple_of(mc * 1024, 1024)
            lhs = kvec_ref[0:1, pl.ds(cs, 1024)]
            rhs = sup_ref[pl.ds(cs, 1024), :]
            return cnt + jax.lax.dot_general(
                lhs, rhs, (((1,), (0,)), ((), ())),
                preferred_element_type=jnp.float32)

        cnt = jax.lax.fori_loop(0, 4, mstep,
                                jnp.zeros((1, _PRE), jnp.float32))
        return jnp.where(cnt < 0.5, initf, 0.0)

    def cond(carry):
        lf, uf = carry
        return jnp.any(lf != uf)

    def step(carry):
        lf, uf = carry
        uf2 = F(lf)
        return F(uf2), uf2

    u0 = initf
    l0 = F(u0)
    lf, uf = jax.lax.while_loop(cond, step, (l0, u0))
    keep = uf > 0.5
    kept_scores = jnp.where(keep, scores_row, -1.0)

    keepf = jnp.where(keep, 1.0, 0.0)
    csum = _cumsum_lanes(keepf)
    total = csum[:, _PRE - 1:_PRE]
    lanef = lane.astype(jnp.float32)
    pos = jnp.where(keep, csum - 1.0, total + lanef - csum)

    posi = pos.astype(jnp.int32)
    pad = jnp.zeros((128, _OUT_COLS - 5), jnp.float32)
    for oc in range(_OUT_ROWS // 128):
        rows = oc * 128 + jax.lax.broadcasted_iota(jnp.int32, (128, 1), 0)
        onehot = jnp.where(rows == posi, 1.0, 0.0)
        ob = jax.lax.dot_general(
            onehot, boxes_col, (((1,), (0,)), ((), ())),
            preferred_element_type=jnp.float32)
        os_ = jnp.sum(onehot * kept_scores, axis=1, keepdims=True)
        out_ref[oc * 128:(oc + 1) * 128, :] = jnp.concatenate(
            [ob, os_, pad], axis=1)


def _nms_core(boxes_col, boxes_row, scores_row):
    sup = pl.pallas_call(
        _nms_build_body,
        out_shape=jax.ShapeDtypeStruct((_PRE, _PRE), jnp.bfloat16),
    )(boxes_col, boxes_row)
    return pl.pallas_call(
        _nms_iter_body,
        out_shape=jax.ShapeDtypeStruct((_OUT_ROWS, _OUT_COLS), jnp.float32),
        scratch_shapes=[pltpu.VMEM((1, _PRE), jnp.bfloat16)],
    )(sup, boxes_col, scores_row)


def kernel(batch_box_preds, batch_cls_preds):
    n = batch_box_preds.shape[2]
    boxes = batch_box_preds[0].T
    scores = jnp.max(batch_cls_preds[0], axis=0)
    bc = jnp.pad(boxes, ((0, _NP - n), (0, 0)))
    sc = jnp.pad(scores, (0, _NP - n), constant_values=-2.0)
    pos, msr = _select_core(sc[:, None], sc[None, :])
    idxf = jnp.arange(_NP, dtype=jnp.float32)[:, None]
    d = jnp.concatenate(
        [bc, msr.reshape(_NP, 1), idxf, jnp.zeros((_NP, 122), jnp.float32)],
        axis=1)
    ci = _sc_compact(pos, d)
    rank2 = _rank2_core(ci, ci[:, 4].reshape(1, _PRE),
                        ci[:, 5].reshape(1, _PRE))
    c = _sc_sort(rank2, ci)
    top_boxes = c[:, :4]
    out = _nms_core(top_boxes, top_boxes.T, c[:, 4][None, :])
    return out[:_POST, :6]


# speedup vs baseline: 4.7156x; 1.0057x over previous
import functools

import jax
import jax.numpy as jnp
from jax import lax
from jax.experimental import pallas as pl
from jax.experimental.pallas import tpu as pltpu
from jax.experimental.pallas import tpu_sc as plsc

_POST = 500
_PRE = 4096
_IOU_T = 0.7
_SCORE_T = 0.1
_BLK = 128
_NBLK = _PRE // _BLK
_OUT_ROWS = 512
_OUT_COLS = 8
_NP = 20480
_NW = 32
_CJUNK = _PRE + _NW
_KLO = -1073741826
_KHI = 1065353216


def _fkey(s):
    k = jax.lax.bitcast_convert_type(s, jnp.int32)
    return jnp.where(k < 0, k ^ jnp.int32(0x7FFFFFFF), k)


def _shift_right_lanes(x, d):
    return jnp.concatenate([jnp.zeros((x.shape[0], d), x.dtype), x[:, :-d]],
                           axis=1)


def _cumsum_lanes(x):
    n = x.shape[1]
    d = 1
    while d < n:
        x = x + _shift_right_lanes(x, d)
        d *= 2
    return x


def _select_body(scol_ref, sraw_ref, pos_ref, msr_ref):
    sraw = sraw_ref[0:1, :]
    msr = jnp.where(sraw > _SCORE_T, sraw, -1.0)
    kr = _fkey(msr)

    cchunk = 2048
    ones_row = jnp.ones((1, cchunk), jnp.float32)
    m = jax.lax.broadcasted_iota(jnp.int32, (1, 128), 1)
    lo = jnp.full((), _KLO, jnp.int32)
    hi = jnp.full((), _KHI, jnp.int32)
    for _ in range(5):
        step = jnp.maximum(jnp.int32(1), (hi - lo + 127) // 128)
        t = jnp.minimum(lo + (m + 1) * step, hi)

        def cstep(jc, acc):
            scol = scol_ref[pl.ds(jc * cchunk, cchunk), 0:1]
            mscol = jnp.where(scol > _SCORE_T, scol, -1.0)
            kcol = _fkey(mscol)
            cmpf = jnp.where(kcol > t, 1.0, 0.0)
            return acc + jax.lax.dot_general(
                ones_row, cmpf, (((1,), (0,)), ((), ())),
                preferred_element_type=jnp.float32)

        counts = jax.lax.fori_loop(0, _NP // cchunk, cstep,
                                   jnp.zeros((1, 128), jnp.float32))
        selm = counts >= float(_PRE)
        lo = jnp.max(jnp.where(selm, t, lo))
        hi = jnp.min(jnp.where(selm, hi, t))
    vstar = hi

    gt = kr > vstar
    ngt = jnp.sum(jnp.where(gt, 1.0, 0.0))
    tief = jnp.where(kr == vstar, 1.0, 0.0)
    texcl = _cumsum_lanes(tief) - tief
    sel = gt | ((kr == vstar) & (ngt + texcl < float(_PRE)))
    self_ = jnp.where(sel, 1.0, 0.0)
    sexcl = _cumsum_lanes(self_) - self_
    pos = jnp.where(sel, sexcl, float(_PRE))
    pos_ref[0:1, :] = pos.astype(jnp.int32)
    msr_ref[0:1, :] = msr


def _select_core(scol, sraw_row):
    return pl.pallas_call(
        _select_body,
        out_shape=[jax.ShapeDtypeStruct((1, _NP), jnp.int32),
                   jax.ShapeDtypeStruct((1, _NP), jnp.float32)],
    )(scol, sraw_row)


def _rank2_body(c_ref, msrow_ref, idxrow_ref, rank_ref):
    krow = _fkey(msrow_ref[0:1, :])
    idxrow = idxrow_ref[0:1, :]

    def step(jc, acc):
        sc = c_ref[pl.ds(jc * _BLK, _BLK), 4:5]
        kc = _fkey(sc)
        idxc = c_ref[pl.ds(jc * _BLK, _BLK), 5:6]
        prec = (kc > krow) | ((kc == krow) & (idxc < idxrow))
        return acc + jnp.sum(jnp.where(prec, 1.0, 0.0), axis=0,
                             keepdims=True)

    rank = jax.lax.fori_loop(0, _NBLK, step,
                             jnp.zeros((1, _PRE), jnp.float32))
    rank_ref[0:1, :] = rank.astype(jnp.int32)


def _rank2_core(c, msrow, idxrow):
    return pl.pallas_call(
        _rank2_body,
        out_shape=jax.ShapeDtypeStruct((1, _PRE), jnp.int32),
    )(c, msrow, idxrow)


@functools.lru_cache(maxsize=None)
def _make_sc_scatter(nrows):
    rpw = nrows // _NW
    nrow = rpw // 128

    def body(rank_hbm, d_hbm, c_hbm, idx_v, d_v, sem):
        wid = lax.axis_index("s") * 2 + lax.axis_index("c")
        pltpu.sync_copy(rank_hbm.at[wid], idx_v)
        pltpu.sync_copy(d_hbm.at[pl.ds(wid * rpw, rpw)], d_v)
        junk = _PRE + wid
        for r in range(nrow):
            for cc in range(8):
                sl = idx_v[r, pl.ds(cc * 16, 16)]
                idx_v[r, pl.ds(cc * 16, 16)] = jnp.where(sl < _PRE, sl, junk)
        for r in range(nrow):
            pltpu.async_copy(d_v.at[pl.ds(r * 128, 128)],
                             c_hbm.at[idx_v.at[r]], sem).wait()

    mesh = plsc.VectorSubcoreMesh(core_axis_name="c", subcore_axis_name="s")
    k = functools.partial(
        pl.kernel,
        mesh=mesh,
        out_type=jax.ShapeDtypeStruct((_CJUNK, 128), jnp.float32),
        scratch_types=[
            pltpu.VMEM((nrow, 128), jnp.int32),
            pltpu.VMEM((rpw, 128), jnp.float32),
            pltpu.SemaphoreType.DMA,
        ],
    )(body)

    def run(rank_row, d):
        return k(rank_row.reshape(_NW, nrow, 128), d)[:_PRE]

    return run


def _sc_compact(rank_row, d):
    return _make_sc_scatter(_NP)(rank_row, d)


def _sc_sort(rank_row, d):
    return _make_sc_scatter(_PRE)(rank_row, d)


def _nms_build_body(boxes_col_ref, boxes_row_ref, sup_ref):
    x1r = boxes_row_ref[0:1, :]
    y1r = boxes_row_ref[1:2, :]
    x2r = boxes_row_ref[2:3, :]
    y2r = boxes_row_ref[3:4, :]
    area_row = jnp.clip(x2r - x1r, 0.0) * jnp.clip(y2r - y1r, 0.0)
    lane = jax.lax.broadcasted_iota(jnp.int32, (1, _PRE), 1)
    bb = 128

    def build(kb, _):
        cs = kb * bb
        bx1 = boxes_col_ref[pl.ds(cs, bb), 0:1]
        by1 = boxes_col_ref[pl.ds(cs, bb), 1:2]
        bx2 = boxes_col_ref[pl.ds(cs, bb), 2:3]
        by2 = boxes_col_ref[pl.ds(cs, bb), 3:4]
        barea = jnp.clip(bx2 - bx1, 0.0) * jnp.clip(by2 - by1, 0.0)

        xx1 = jnp.maximum(bx1, x1r)
        yy1 = jnp.maximum(by1, y1r)
        xx2 = jnp.minimum(bx2, x2r)
        yy2 = jnp.minimum(by2, y2r)
        inter = jnp.clip(xx2 - xx1, 0.0) * jnp.clip(yy2 - yy1, 0.0)
        union = barea + area_row - inter
        iou = inter / jnp.maximum(union, 1e-8)
        gj = cs + jax.lax.broadcasted_iota(jnp.int32, (bb, 1), 0)
        sup = (iou > _IOU_T) & (gj < lane)
        sup_ref[pl.ds(cs, bb), :] = jnp.where(sup, 1.0, 0.0).astype(
            jnp.bfloat16)
        return 0

    jax.lax.fori_loop(0, _PRE // bb, build, 0)


def _nms_iter_body(sup_ref, boxes_col_ref, scores_row_ref, out_ref, kvec_ref):
    scores_row = scores_row_ref[0:1, :]
    lane = jax.lax.broadcasted_iota(jnp.int32, (1, _PRE), 1)
    init = scores_row > _SCORE_T
    initf = jnp.where(init, 1.0, 0.0)
    boxes_col = boxes_col_ref[:, :]

    def F(kf):
        kvec_ref[0:1, :] = kf.astype(jnp.bfloat16)

        def mstep(mc, cnt):
            cs = pl.multiple_of(mc * 2048, 2048)
            lhs = kvec_ref[0:1, pl.ds(cs, 2048)]
            rhs = sup_ref[pl.ds(cs, 2048), :]
            return cnt + jax.lax.dot_general(
                lhs, rhs, (((1,), (0,)), ((), ())),
                preferred_element_type=jnp.float32)

        cnt = jax.lax.fori_loop(0, 2, mstep,
                                jnp.zeros((1, _PRE), jnp.float32))
        return jnp.where(cnt < 0.5, initf, 0.0)

    def cond(carry):
        lf, uf = carry
        return jnp.any(lf != uf)

    def step(carry):
        lf, uf = carry
        uf2 = F(lf)
        return F(uf2), uf2

    u0 = initf
    l0 = F(u0)
    lf, uf = jax.lax.while_loop(cond, step, (l0, u0))
    keep = uf > 0.5
    kept_scores = jnp.where(keep, scores_row, -1.0)

    keepf = jnp.where(keep, 1.0, 0.0)
    csum = _cumsum_lanes(keepf)
    total = csum[:, _PRE - 1:_PRE]
    lanef = lane.astype(jnp.float32)
    pos = jnp.where(keep, csum - 1.0, total + lanef - csum)

    posi = pos.astype(jnp.int32)
    pad = jnp.zeros((128, _OUT_COLS - 5), jnp.float32)
    for oc in range(_OUT_ROWS // 128):
        rows = oc * 128 + jax.lax.broadcasted_iota(jnp.int32, (128, 1), 0)
        onehot = jnp.where(rows == posi, 1.0, 0.0)
        ob = jax.lax.dot_general(
            onehot, boxes_col, (((1,), (0,)), ((), ())),
            preferred_element_type=jnp.float32)
        os_ = jnp.sum(onehot * kept_scores, axis=1, keepdims=True)
        out_ref[oc * 128:(oc + 1) * 128, :] = jnp.concatenate(
            [ob, os_, pad], axis=1)


def _nms_core(boxes_col, boxes_row, scores_row):
    sup = pl.pallas_call(
        _nms_build_body,
        out_shape=jax.ShapeDtypeStruct((_PRE, _PRE), jnp.bfloat16),
    )(boxes_col, boxes_row)
    return pl.pallas_call(
        _nms_iter_body,
        out_shape=jax.ShapeDtypeStruct((_OUT_ROWS, _OUT_COLS), jnp.float32),
        scratch_shapes=[pltpu.VMEM((1, _PRE), jnp.bfloat16)],
    )(sup, boxes_col, scores_row)


def kernel(batch_box_preds, batch_cls_preds):
    n = batch_box_preds.shape[2]
    boxes = batch_box_preds[0].T
    scores = jnp.max(batch_cls_preds[0], axis=0)
    bc = jnp.pad(boxes, ((0, _NP - n), (0, 0)))
    sc = jnp.pad(scores, (0, _NP - n), constant_values=-2.0)
    pos, msr = _select_core(sc[:, None], sc[None, :])
    idxf = jnp.arange(_NP, dtype=jnp.float32)[:, None]
    d = jnp.concatenate(
        [bc, msr.reshape(_NP, 1), idxf, jnp.zeros((_NP, 122), jnp.float32)],
        axis=1)
    ci = _sc_compact(pos, d)
    rank2 = _rank2_core(ci, ci[:, 4].reshape(1, _PRE),
                        ci[:, 5].reshape(1, _PRE))
    c = _sc_sort(rank2, ci)
    top_boxes = c[:, :4]
    out = _nms_core(top_boxes, top_boxes.T, c[:, 4][None, :])
    return out[:_POST, :6]
